# baseline (speedup 1.0000x reference)
import functools

import jax
import jax.numpy as jnp
from jax import lax
from jax.experimental import pallas as pl
from jax.experimental.pallas import tpu as pltpu

_TILE_V = 2048


def _body(stok_ref, order_ref, starts_ref, tbl_ref, w_ref, b_ref, o_ref,
          x_ref, tpose_ref, sem, *, tile_v):
    p = pl.program_id(0)
    j = pl.program_id(1)

    @pl.when(p == 0)
    def _gather():
        tpose_ref[...] = jnp.transpose(tbl_ref[...])
        lo = j * tile_v

        def issue_one(i, carry):
            local = stok_ref[i] - lo
            dst = order_ref[i]
            pltpu.make_async_copy(
                tpose_ref.at[pl.ds(local, 1), :],
                x_ref.at[pl.ds(dst, 1), :],
                sem,
            ).start()
            return carry

        def drain_one(i, carry):
            pltpu.make_async_copy(
                tpose_ref.at[pl.ds(0, 1), :],
                x_ref.at[pl.ds(0, 1), :],
                sem,
            ).wait()
            return carry

        lax.fori_loop(starts_ref[j], starts_ref[j + 1], issue_one, 0)
        lax.fori_loop(starts_ref[j], starts_ref[j + 1], drain_one, 0)

    @pl.when(p == 1)
    def _matmul():
        acc = lax.dot_general(
            w_ref[...],
            x_ref[...],
            (((0,), (1,)), ((), ())),
            preferred_element_type=jnp.float32,
        )
        o_ref[...] = acc + jnp.transpose(b_ref[...])


def kernel(input_token, emb_table, fc_weight, fc_bias):
    V, D = emb_table.shape
    B = input_token.shape[0]
    tile_v = _TILE_V
    grid_j = pl.cdiv(V, tile_v)

    tokens = input_token.astype(jnp.int32)
    order = jnp.argsort(tokens).astype(jnp.int32)
    sorted_tok = jnp.take(tokens, order)
    shift = tile_v.bit_length() - 1
    counts = jnp.zeros(grid_j, jnp.int32).at[tokens >> shift].add(1)
    starts = jnp.concatenate(
        [jnp.zeros(1, jnp.int32), jnp.cumsum(counts, dtype=jnp.int32)]
    )

    table_t = emb_table.T
    w_t = fc_weight.T
    bias2d = fc_bias.reshape(1, V)

    grid_spec = pltpu.PrefetchScalarGridSpec(
        num_scalar_prefetch=3,
        grid=(2, grid_j),
        in_specs=[
            pl.BlockSpec(
                (D, tile_v), lambda p, j, *_: (0, jnp.where(p == 0, j, 0))
            ),
            pl.BlockSpec(
                (D, tile_v), lambda p, j, *_: (0, jnp.where(p == 1, j, 0))
            ),
            pl.BlockSpec(
                (1, tile_v), lambda p, j, *_: (0, jnp.where(p == 1, j, 0))
            ),
        ],
        out_specs=pl.BlockSpec(
            (tile_v, B), lambda p, j, *_: (jnp.where(p == 1, j, 0), 0)
        ),
        scratch_shapes=[
            pltpu.VMEM((B, D), jnp.float32),
            pltpu.VMEM((tile_v, D), jnp.float32),
            pltpu.SemaphoreType.DMA,
        ],
    )
    out_t = pl.pallas_call(
        functools.partial(_body, tile_v=tile_v),
        grid_spec=grid_spec,
        out_shape=jax.ShapeDtypeStruct((V, B), jnp.float32),
        compiler_params=pltpu.CompilerParams(
            dimension_semantics=("arbitrary", "arbitrary"),
        ),
    )(sorted_tok, order, starts, table_t, w_t, bias2d)
    return out_t.T


# speedup vs baseline: 1.1435x; 1.1435x over previous
import functools

import jax
import jax.numpy as jnp
from jax import lax
from jax.experimental import pallas as pl
from jax.experimental.pallas import tpu as pltpu

_TILE_V = 2048


def _body(stok_ref, order_ref, starts_ref, tbl_ref, w_ref, b_ref, o_ref,
          x_ref, tpose_ref, sem, *, tile_v):
    p = pl.program_id(0)
    j = pl.program_id(1)

    @pl.when(p == 0)
    def _gather():
        d = tbl_ref.shape[0]
        eye = (
            lax.broadcasted_iota(jnp.int32, (d, d), 0)
            == lax.broadcasted_iota(jnp.int32, (d, d), 1)
        ).astype(jnp.float32)
        tpose_ref[...] = lax.dot_general(
            tbl_ref[...], eye, (((0,), (0,)), ((), ())),
            preferred_element_type=jnp.float32,
        )
        lo = j * tile_v

        def copy_one(i, carry):
            local = stok_ref[i] - lo
            dst = order_ref[i]
            d = pltpu.make_async_copy(
                tpose_ref.at[pl.ds(local, 1), :],
                x_ref.at[pl.ds(dst, 1), :],
                sem,
            )
            d.start()
            d.wait()
            return carry

        lax.fori_loop(starts_ref[j], starts_ref[j + 1], copy_one, 0)

    @pl.when(p == 1)
    def _matmul():
        acc = lax.dot_general(
            w_ref[...],
            x_ref[...],
            (((0,), (1,)), ((), ())),
            preferred_element_type=jnp.float32,
        )
        o_ref[...] = acc + jnp.transpose(b_ref[...])


def kernel(input_token, emb_table, fc_weight, fc_bias):
    V, D = emb_table.shape
    B = input_token.shape[0]
    tile_v = _TILE_V
    grid_j = pl.cdiv(V, tile_v)

    tokens = input_token.astype(jnp.int32)
    sorted_tok, order = lax.sort(
        (tokens, jnp.arange(B, dtype=jnp.int32)), num_keys=1
    )
    bounds = jnp.arange(grid_j + 1, dtype=jnp.int32) * tile_v
    starts = jnp.sum(
        tokens[None, :] < bounds[:, None], axis=1, dtype=jnp.int32
    )

    table_t = emb_table.T
    w_t = fc_weight.T
    bias2d = fc_bias.reshape(1, V)

    grid_spec = pltpu.PrefetchScalarGridSpec(
        num_scalar_prefetch=3,
        grid=(2, grid_j),
        in_specs=[
            pl.BlockSpec(
                (D, tile_v), lambda p, j, *_: (0, jnp.where(p == 0, j, 0))
            ),
            pl.BlockSpec(
                (D, tile_v), lambda p, j, *_: (0, jnp.where(p == 1, j, 0))
            ),
            pl.BlockSpec(
                (1, tile_v), lambda p, j, *_: (0, jnp.where(p == 1, j, 0))
            ),
        ],
        out_specs=pl.BlockSpec(
            (tile_v, B), lambda p, j, *_: (jnp.where(p == 1, j, 0), 0)
        ),
        scratch_shapes=[
            pltpu.VMEM((B, D), jnp.float32),
            pltpu.VMEM((tile_v, D), jnp.float32),
            pltpu.SemaphoreType.DMA,
        ],
    )
    out_t = pl.pallas_call(
        functools.partial(_body, tile_v=tile_v),
        grid_spec=grid_spec,
        out_shape=jax.ShapeDtypeStruct((V, B), jnp.float32),
        compiler_params=pltpu.CompilerParams(
            dimension_semantics=("arbitrary", "arbitrary"),
        ),
    )(sorted_tok, order, starts, table_t, w_t, bias2d)
    return out_t.T


# speedup vs baseline: 1.1715x; 1.0244x over previous
import functools

import jax
import jax.numpy as jnp
from jax import lax
from jax.experimental import pallas as pl
from jax.experimental.pallas import tpu as pltpu

_TILE_V = 2048
_CHUNK = 32


def _body(starts_ref, stok_ref, order_ref, tbl_ref, w_ref, b_ref, o_ref,
          xs_ref, x_ref, stokc_ref, *, tile_v, batch, v_total):
    p = pl.program_id(0)
    j = pl.program_id(1)

    @pl.when((p == 0) & (j == 0))
    def _init():
        xs_ref[...] = jnp.zeros(xs_ref.shape, xs_ref.dtype)
        stokc_ref[...] = jnp.transpose(stok_ref[...])

    @pl.when(p == 0)
    def _gather():
        lo = j * tile_v
        start = starts_ref[j]
        end = starts_ref[j + 1]
        d, _ = tbl_ref.shape
        colmask = (
            lax.broadcasted_iota(jnp.int32, (d, tile_v), 1) < v_total - lo
        )
        tbl = jnp.where(colmask, tbl_ref[...], 0.0)

        def do_chunk(k, carry):
            base = k * _CHUNK
            toks = stokc_ref[pl.ds(base, _CHUNK), :]
            local = toks - lo
            onehot = (
                local
                == lax.broadcasted_iota(jnp.int32, (_CHUNK, tile_v), 1)
            ).astype(jnp.float32)
            contrib = lax.dot_general(
                onehot, tbl, (((1,), (1,)), ((), ())),
                preferred_element_type=jnp.float32,
            )
            xs_ref[pl.ds(base, _CHUNK), :] = (
                xs_ref[pl.ds(base, _CHUNK), :] + contrib
            )
            return carry

        @pl.when(end > start)
        def _():
            lax.fori_loop(
                start // _CHUNK, (end - 1) // _CHUNK + 1, do_chunk, 0
            )

    @pl.when((p == 1) & (j == 0))
    def _unpermute():
        perm = (
            jnp.broadcast_to(order_ref[...], (batch, batch))
            == lax.broadcasted_iota(jnp.int32, (batch, batch), 0)
        ).astype(jnp.float32)
        x_ref[...] = lax.dot_general(
            perm, xs_ref[...], (((1,), (0,)), ((), ())),
            preferred_element_type=jnp.float32,
        )

    @pl.when(p == 1)
    def _matmul():
        acc = lax.dot_general(
            w_ref[...],
            x_ref[...],
            (((0,), (1,)), ((), ())),
            preferred_element_type=jnp.float32,
        )
        o_ref[...] = acc + jnp.transpose(b_ref[...])


def kernel(input_token, emb_table, fc_weight, fc_bias):
    V, D = emb_table.shape
    B = input_token.shape[0]
    tile_v = _TILE_V
    grid_j = pl.cdiv(V, tile_v)

    tokens = input_token.astype(jnp.int32)
    sorted_tok, order = lax.sort(
        (tokens, jnp.arange(B, dtype=jnp.int32)), num_keys=1
    )
    bounds = jnp.arange(grid_j + 1, dtype=jnp.int32) * tile_v
    starts = jnp.sum(
        tokens[None, :] < bounds[:, None], axis=1, dtype=jnp.int32
    )

    table_t = emb_table.T
    w_t = fc_weight.T
    bias2d = fc_bias.reshape(1, V)

    grid_spec = pltpu.PrefetchScalarGridSpec(
        num_scalar_prefetch=1,
        grid=(2, grid_j),
        in_specs=[
            pl.BlockSpec((1, B), lambda p, j, *_: (0, 0)),
            pl.BlockSpec((1, B), lambda p, j, *_: (0, 0)),
            pl.BlockSpec(
                (D, tile_v), lambda p, j, *_: (0, jnp.where(p == 0, j, 0))
            ),
            pl.BlockSpec(
                (D, tile_v), lambda p, j, *_: (0, jnp.where(p == 1, j, 0))
            ),
            pl.BlockSpec(
                (1, tile_v), lambda p, j, *_: (0, jnp.where(p == 1, j, 0))
            ),
        ],
        out_specs=pl.BlockSpec(
            (tile_v, B), lambda p, j, *_: (jnp.where(p == 1, j, 0), 0)
        ),
        scratch_shapes=[
            pltpu.VMEM((B, D), jnp.float32),
            pltpu.VMEM((B, D), jnp.float32),
            pltpu.VMEM((B, 1), jnp.int32),
        ],
    )
    out_t = pl.pallas_call(
        functools.partial(_body, tile_v=tile_v, batch=B, v_total=V),
        grid_spec=grid_spec,
        out_shape=jax.ShapeDtypeStruct((V, B), jnp.float32),
        compiler_params=pltpu.CompilerParams(
            dimension_semantics=("arbitrary", "arbitrary"),
        ),
    )(
        starts,
        sorted_tok.reshape(1, B),
        order.reshape(1, B),
        table_t,
        w_t,
        bias2d,
    )
    return out_t.T
